# Initial kernel scaffold; baseline (speedup 1.0000x reference)
import jax
import jax.numpy as jnp
from jax.experimental import pallas as pl


def kernel(Q, K, V):
    raise NotImplementedError("write your pallas kernel here")


# speedup vs baseline: 3.6800x; 3.6800x over previous
import functools
import math

import numpy as np
import jax
import jax.numpy as jnp
from jax import lax
from jax.experimental import pallas as pl
from jax.experimental.pallas import tpu as pltpu
from jax.experimental.pallas import tpu_sc as plsc

_FACTOR = 1


@functools.lru_cache(maxsize=None)
def _dft_mats(L: int, FP: int):
    F = L // 2 + 1
    f = np.arange(FP, dtype=np.int64)[:, None]
    t = np.arange(L, dtype=np.int64)[None, :]
    ang = 2.0 * np.pi * ((f * t) % L).astype(np.float64) / L
    valid = (f < F).astype(np.float64)
    cr = (np.cos(ang) * valid).astype(np.float32)
    ci = (-np.sin(ang) * valid).astype(np.float32)
    alpha = np.where((f == 0) | (f == L // 2), 1.0, 2.0) * valid
    dr = ((np.cos(ang) * alpha / L).T).astype(np.float32)
    di = ((-np.sin(ang) * alpha / L).T).astype(np.float32)
    return cr, ci, dr, di


def _corr_topk_body(top_k, L, CB, NFB, cr, ci, dr, di, qt, kt,
                    w_out, d_out, corr_acc):
    hi = lax.Precision.HIGHEST
    f32 = jnp.float32
    fb = pl.program_id(1)
    q = qt[...]
    k = kt[...]
    crv = cr[...]
    civ = ci[...]
    qr = jnp.dot(crv, q, preferred_element_type=f32, precision=hi)
    qi = jnp.dot(civ, q, preferred_element_type=f32, precision=hi)
    kr = jnp.dot(crv, k, preferred_element_type=f32, precision=hi)
    ki = jnp.dot(civ, k, preferred_element_type=f32, precision=hi)
    rr = qr * kr + qi * ki
    ri = qi * kr - qr * ki
    part = (jnp.dot(dr[...], rr, preferred_element_type=f32, precision=hi)
            + jnp.dot(di[...], ri, preferred_element_type=f32, precision=hi))

    @pl.when(fb == 0)
    def _():
        corr_acc[...] = part

    @pl.when(fb > 0)
    def _():
        corr_acc[...] = corr_acc[...] + part

    @pl.when(fb == NFB - 1)
    def _():
        iot = lax.broadcasted_iota(jnp.int32, (L, CB), 0)
        c = corr_acc[...]
        ws, ds = [], []
        for _i in range(top_k):
            mx = jnp.max(c, axis=0, keepdims=True)
            eq = c >= mx
            idx = jnp.min(jnp.where(eq, iot, L), axis=0, keepdims=True)
            ws.append(mx)
            ds.append(idx)
            c = jnp.where(iot == idx, f32(-3.0e38), c)
        w = jnp.concatenate(ws, axis=0)
        d = jnp.concatenate(ds, axis=0)
        m = jnp.max(w, axis=0, keepdims=True)
        e = jnp.exp(w - m)
        w = e / jnp.sum(e, axis=0, keepdims=True)
        pad = 16 - top_k
        w_out[...] = jnp.concatenate([w, jnp.zeros((pad, CB), f32)], axis=0)
        d_out[...] = jnp.concatenate(
            [d, jnp.zeros((pad, CB), jnp.int32)], axis=0)


def _corr_topk(qt, kt, top_k, CB=128, FP=1152, FB=384, interpret=False):
    L, C = qt.shape
    cr, ci, dr, di = _dft_mats(L, FP)
    NFB = FP // FB
    body = functools.partial(_corr_topk_body, top_k, L, CB, NFB)
    grid = (C // CB, NFB)
    w16, d16 = pl.pallas_call(
        body,
        grid=grid,
        in_specs=[
            pl.BlockSpec((FB, L), lambda j, fb: (fb, 0)),
            pl.BlockSpec((FB, L), lambda j, fb: (fb, 0)),
            pl.BlockSpec((L, FB), lambda j, fb: (0, fb)),
            pl.BlockSpec((L, FB), lambda j, fb: (0, fb)),
            pl.BlockSpec((L, CB), lambda j, fb: (0, j)),
            pl.BlockSpec((L, CB), lambda j, fb: (0, j)),
        ],
        out_specs=[
            pl.BlockSpec((16, CB), lambda j, fb: (0, j)),
            pl.BlockSpec((16, CB), lambda j, fb: (0, j)),
        ],
        out_shape=[
            jax.ShapeDtypeStruct((16, C), jnp.float32),
            jax.ShapeDtypeStruct((16, C), jnp.int32),
        ],
        scratch_shapes=[pltpu.VMEM((L, CB), jnp.float32)],
        interpret=interpret,
    )(jnp.asarray(cr), jnp.asarray(ci), jnp.asarray(dr), jnp.asarray(di),
      qt, kt)
    return w16, d16


def _sc_agg(vt, wt, dt, top_k):
    C, L = vt.shape
    info = plsc.get_sparse_core_info()
    nw = info.num_cores * info.num_subcores
    cols_per = C // nw
    mesh = plsc.VectorSubcoreMesh(core_axis_name="c", subcore_axis_name="s")

    @functools.partial(
        pl.kernel,
        out_type=jax.ShapeDtypeStruct((C, L), jnp.float32),
        mesh=mesh,
        scratch_types=[
            pltpu.VMEM((2 * L,), jnp.float32),
            pltpu.VMEM((L,), jnp.float32),
            pltpu.VMEM((16,), jnp.float32),
            pltpu.VMEM((16,), jnp.int32),
        ],
    )
    def body(vt_hbm, wt_hbm, dt_hbm, out_hbm, vbuf, obuf, wbuf, dbuf):
        wid = lax.axis_index("s") * info.num_cores + lax.axis_index("c")

        def col_body(jj, carry):
            c0 = wid * cols_per + jj
            pltpu.sync_copy(vt_hbm.at[c0], vbuf.at[pl.ds(0, L)])
            pltpu.sync_copy(vt_hbm.at[c0], vbuf.at[pl.ds(L, L)])
            pltpu.sync_copy(wt_hbm.at[c0], wbuf)
            pltpu.sync_copy(dt_hbm.at[c0], dbuf)
            wv = wbuf[...]
            dv = dbuf[...]
            wss = [wv[i] for i in range(top_k)]
            dss = [dv[i] for i in range(top_k)]

            def vec_body(v, carry2):
                base = v * 16
                acc = wss[0] * vbuf[pl.ds(base + dss[0], 16)]
                for i in range(1, top_k):
                    acc = acc + wss[i] * vbuf[pl.ds(base + dss[i], 16)]
                obuf[pl.ds(base, 16)] = acc
                return carry2

            lax.fori_loop(0, L // 16, vec_body, 0)
            pltpu.sync_copy(obuf, out_hbm.at[c0])
            return carry

        lax.fori_loop(0, cols_per, col_body, 0)

    return body(vt, wt, dt)


def kernel(Q, K, V):
    B, H, L, D = Q.shape
    C = B * H * D
    top_k = int(_FACTOR * math.log(L))

    qt = jnp.transpose(Q, (2, 0, 1, 3)).reshape(L, C)
    kt = jnp.transpose(K, (2, 0, 1, 3)).reshape(L, C)
    w16, d16 = _corr_topk(qt, kt, top_k)

    vt = jnp.transpose(V, (0, 1, 3, 2)).reshape(C, L)
    wt = jnp.transpose(w16, (1, 0))
    dt = jnp.transpose(d16, (1, 0))
    out_t = _sc_agg(vt, wt, dt, top_k)

    return jnp.transpose(out_t.reshape(B, H, D, L), (0, 1, 3, 2))


# speedup vs baseline: 5.6018x; 1.5222x over previous
import functools
import math

import numpy as np
import jax
import jax.numpy as jnp
from jax import lax
from jax.experimental import pallas as pl
from jax.experimental.pallas import tpu as pltpu
from jax.experimental.pallas import tpu_sc as plsc

_FACTOR = 1


@functools.lru_cache(maxsize=None)
def _dft_mats(L: int, FP: int):
    F = L // 2 + 1
    f = np.arange(FP, dtype=np.int64)[:, None]
    t = np.arange(L, dtype=np.int64)[None, :]
    ang = 2.0 * np.pi * ((f * t) % L).astype(np.float64) / L
    valid = (f < F).astype(np.float64)
    cr = (np.cos(ang) * valid).astype(np.float32)
    ci = (-np.sin(ang) * valid).astype(np.float32)
    alpha = np.where((f == 0) | (f == L // 2), 1.0, 2.0) * valid
    dr = ((np.cos(ang) * alpha / L).T).astype(np.float32)
    di = ((-np.sin(ang) * alpha / L).T).astype(np.float32)
    return cr, ci, dr, di


def _split_bf16(x):
    hi = x.astype(jnp.bfloat16)
    lo = (x - hi.astype(jnp.float32)).astype(jnp.bfloat16)
    return hi, lo


def _mm3(ah, al, bh, bl):
    f32 = jnp.float32
    return (jnp.dot(ah, bh, preferred_element_type=f32)
            + jnp.dot(ah, bl, preferred_element_type=f32)
            + jnp.dot(al, bh, preferred_element_type=f32))


def _corr_topk_body(top_k, L, CB, NFB, crh, crl, cih, cil, drh, drl,
                    dih, dil, qt, kt, w_out, d_out, corr_acc):
    f32 = jnp.float32
    fb = pl.program_id(1)
    qh, ql = _split_bf16(qt[...])
    kh, kl = _split_bf16(kt[...])
    qr = _mm3(crh[...], crl[...], qh, ql)
    qi = _mm3(cih[...], cil[...], qh, ql)
    kr = _mm3(crh[...], crl[...], kh, kl)
    ki = _mm3(cih[...], cil[...], kh, kl)
    rr = qr * kr + qi * ki
    ri = qi * kr - qr * ki
    rrh, rrl = _split_bf16(rr)
    rih, ril = _split_bf16(ri)
    part = (_mm3(drh[...], drl[...], rrh, rrl)
            + _mm3(dih[...], dil[...], rih, ril))

    @pl.when(fb == 0)
    def _():
        corr_acc[...] = part

    @pl.when(fb > 0)
    def _():
        corr_acc[...] = corr_acc[...] + part

    @pl.when(fb == NFB - 1)
    def _():
        iot = lax.broadcasted_iota(jnp.int32, (L, CB), 0)
        c = corr_acc[...]
        ws, ds = [], []
        for _i in range(top_k):
            mx = jnp.max(c, axis=0, keepdims=True)
            eq = c >= mx
            idx = jnp.min(jnp.where(eq, iot, L), axis=0, keepdims=True)
            ws.append(mx)
            ds.append(idx)
            c = jnp.where(iot == idx, f32(-3.0e38), c)
        w = jnp.concatenate(ws, axis=0)
        d = jnp.concatenate(ds, axis=0)
        m = jnp.max(w, axis=0, keepdims=True)
        e = jnp.exp(w - m)
        w = e / jnp.sum(e, axis=0, keepdims=True)
        pad = 16 - top_k
        w_out[...] = jnp.concatenate([w, jnp.zeros((pad, CB), f32)], axis=0)
        d_out[...] = jnp.concatenate(
            [d, jnp.zeros((pad, CB), jnp.int32)], axis=0)


@functools.lru_cache(maxsize=None)
def _dft_mats_split(L: int, FP: int):
    import ml_dtypes
    out = []
    for m in _dft_mats(L, FP):
        hi = m.astype(ml_dtypes.bfloat16)
        lo = (m - hi.astype(np.float32)).astype(ml_dtypes.bfloat16)
        out.append(hi)
        out.append(lo)
    return tuple(out)


def _corr_topk(qt, kt, top_k, CB=128, FP=1152, FB=384, interpret=False):
    L, C = qt.shape
    mats = _dft_mats_split(L, FP)
    NFB = FP // FB
    body = functools.partial(_corr_topk_body, top_k, L, CB, NFB)
    grid = (C // CB, NFB)
    fwd_spec = pl.BlockSpec((FB, L), lambda j, fb: (fb, 0))
    inv_spec = pl.BlockSpec((L, FB), lambda j, fb: (0, fb))
    w16, d16 = pl.pallas_call(
        body,
        grid=grid,
        in_specs=[fwd_spec] * 4 + [inv_spec] * 4 + [
            pl.BlockSpec((L, CB), lambda j, fb: (0, j)),
            pl.BlockSpec((L, CB), lambda j, fb: (0, j)),
        ],
        out_specs=[
            pl.BlockSpec((16, CB), lambda j, fb: (0, j)),
            pl.BlockSpec((16, CB), lambda j, fb: (0, j)),
        ],
        out_shape=[
            jax.ShapeDtypeStruct((16, C), jnp.float32),
            jax.ShapeDtypeStruct((16, C), jnp.int32),
        ],
        scratch_shapes=[pltpu.VMEM((L, CB), jnp.float32)],
        interpret=interpret,
    )(*[jnp.asarray(m) for m in mats], qt, kt)
    return w16, d16


def _sc_agg(vt, wt, dt, top_k):
    C, L = vt.shape
    info = plsc.get_sparse_core_info()
    nw = info.num_cores * info.num_subcores
    cols_per = C // nw
    mesh = plsc.VectorSubcoreMesh(core_axis_name="c", subcore_axis_name="s")

    @functools.partial(
        pl.kernel,
        out_type=jax.ShapeDtypeStruct((C, L), jnp.float32),
        mesh=mesh,
        scratch_types=[
            pltpu.VMEM((2 * L,), jnp.float32),
            pltpu.VMEM((L,), jnp.float32),
            pltpu.VMEM((16,), jnp.float32),
            pltpu.VMEM((16,), jnp.int32),
        ],
    )
    def body(vt_hbm, wt_hbm, dt_hbm, out_hbm, vbuf, obuf, wbuf, dbuf):
        wid = lax.axis_index("s") * info.num_cores + lax.axis_index("c")

        def col_body(jj, carry):
            c0 = wid * cols_per + jj
            pltpu.sync_copy(vt_hbm.at[c0], vbuf.at[pl.ds(0, L)])
            pltpu.sync_copy(vt_hbm.at[c0], vbuf.at[pl.ds(L, L)])
            pltpu.sync_copy(wt_hbm.at[c0], wbuf)
            pltpu.sync_copy(dt_hbm.at[c0], dbuf)
            wv = wbuf[...]
            dv = dbuf[...]
            wss = [wv[i] for i in range(top_k)]
            dss = [dv[i] for i in range(top_k)]

            def vec_body(v, carry2):
                base = v * 16
                acc = wss[0] * vbuf[pl.ds(base + dss[0], 16)]
                for i in range(1, top_k):
                    acc = acc + wss[i] * vbuf[pl.ds(base + dss[i], 16)]
                obuf[pl.ds(base, 16)] = acc
                return carry2

            lax.fori_loop(0, L // 16, vec_body, 0)
            pltpu.sync_copy(obuf, out_hbm.at[c0])
            return carry

        lax.fori_loop(0, cols_per, col_body, 0)

    return body(vt, wt, dt)


def kernel(Q, K, V):
    B, H, L, D = Q.shape
    C = B * H * D
    top_k = int(_FACTOR * math.log(L))

    qt = jnp.transpose(Q, (2, 0, 1, 3)).reshape(L, C)
    kt = jnp.transpose(K, (2, 0, 1, 3)).reshape(L, C)
    w16, d16 = _corr_topk(qt, kt, top_k)

    vt = jnp.transpose(V, (0, 1, 3, 2)).reshape(C, L)
    wt = jnp.transpose(w16, (1, 0))
    dt = jnp.transpose(d16, (1, 0))
    out_t = _sc_agg(vt, wt, dt, top_k)

    return jnp.transpose(out_t.reshape(B, H, D, L), (0, 1, 3, 2))


# speedup vs baseline: 7.1227x; 1.2715x over previous
import functools
import math

import numpy as np
import jax
import jax.numpy as jnp
from jax import lax
from jax.experimental import pallas as pl
from jax.experimental.pallas import tpu as pltpu
from jax.experimental.pallas import tpu_sc as plsc

_FACTOR = 1


@functools.lru_cache(maxsize=None)
def _dft_mats(L: int, FP: int):
    F = L // 2 + 1
    f = np.arange(FP, dtype=np.int64)[:, None]
    t = np.arange(L, dtype=np.int64)[None, :]
    ang = 2.0 * np.pi * ((f * t) % L).astype(np.float64) / L
    valid = (f < F).astype(np.float64)
    cr = (np.cos(ang) * valid).astype(np.float32)
    ci = (-np.sin(ang) * valid).astype(np.float32)
    alpha = np.where((f == 0) | (f == L // 2), 1.0, 2.0) * valid
    dr = ((np.cos(ang) * alpha / L).T).astype(np.float32)
    di = ((-np.sin(ang) * alpha / L).T).astype(np.float32)
    return cr, ci, dr, di


def _split_bf16(x):
    hi = x.astype(jnp.bfloat16)
    lo = (x - hi.astype(jnp.float32)).astype(jnp.bfloat16)
    return hi, lo


def _mm3(ah, al, bh, bl):
    f32 = jnp.float32
    return (jnp.dot(ah, bh, preferred_element_type=f32)
            + jnp.dot(ah, bl, preferred_element_type=f32)
            + jnp.dot(al, bh, preferred_element_type=f32))


def _corr_topk_body(top_k, L, CB, NFB, crh, crl, cih, cil, drh, drl,
                    dih, dil, qt, kt, w_out, d_out, corr_acc):
    f32 = jnp.float32
    fb = pl.program_id(1)
    qh, ql = _split_bf16(qt[...])
    kh, kl = _split_bf16(kt[...])
    qr = _mm3(crh[...], crl[...], qh, ql)
    qi = _mm3(cih[...], cil[...], qh, ql)
    kr = _mm3(crh[...], crl[...], kh, kl)
    ki = _mm3(cih[...], cil[...], kh, kl)
    rr = qr * kr + qi * ki
    ri = qi * kr - qr * ki
    rrh, rrl = _split_bf16(rr)
    rih, ril = _split_bf16(ri)
    part = (_mm3(drh[...], drl[...], rrh, rrl)
            + _mm3(dih[...], dil[...], rih, ril))

    @pl.when(fb == 0)
    def _():
        corr_acc[...] = part

    @pl.when(fb > 0)
    def _():
        corr_acc[...] = corr_acc[...] + part

    @pl.when(fb == NFB - 1)
    def _():
        iot = lax.broadcasted_iota(jnp.int32, (L, CB), 0)
        c = corr_acc[...]
        ws, ds = [], []
        for _i in range(top_k):
            mx = jnp.max(c, axis=0, keepdims=True)
            eq = c >= mx
            idx = jnp.min(jnp.where(eq, iot, L), axis=0, keepdims=True)
            ws.append(mx)
            ds.append(idx)
            c = jnp.where(iot == idx, f32(-3.0e38), c)
        w = jnp.concatenate(ws, axis=0)
        d = jnp.concatenate(ds, axis=0)
        m = jnp.max(w, axis=0, keepdims=True)
        e = jnp.exp(w - m)
        w = e / jnp.sum(e, axis=0, keepdims=True)
        pad = 16 - top_k
        w_out[...] = jnp.concatenate([w, jnp.zeros((pad, CB), f32)], axis=0)
        d_out[...] = jnp.concatenate(
            [d, jnp.zeros((pad, CB), jnp.int32)], axis=0)


@functools.lru_cache(maxsize=None)
def _dft_mats_split(L: int, FP: int):
    import ml_dtypes
    out = []
    for m in _dft_mats(L, FP):
        hi = m.astype(ml_dtypes.bfloat16)
        lo = (m - hi.astype(np.float32)).astype(ml_dtypes.bfloat16)
        out.append(hi)
        out.append(lo)
    return tuple(out)


def _corr_topk(qt, kt, top_k, CB=128, FP=1152, FB=384, interpret=False):
    L, C = qt.shape
    mats = _dft_mats_split(L, FP)
    NFB = FP // FB
    body = functools.partial(_corr_topk_body, top_k, L, CB, NFB)
    grid = (C // CB, NFB)
    fwd_spec = pl.BlockSpec((FB, L), lambda j, fb: (fb, 0))
    inv_spec = pl.BlockSpec((L, FB), lambda j, fb: (0, fb))
    w16, d16 = pl.pallas_call(
        body,
        grid=grid,
        in_specs=[fwd_spec] * 4 + [inv_spec] * 4 + [
            pl.BlockSpec((L, CB), lambda j, fb: (0, j)),
            pl.BlockSpec((L, CB), lambda j, fb: (0, j)),
        ],
        out_specs=[
            pl.BlockSpec((16, CB), lambda j, fb: (0, j)),
            pl.BlockSpec((16, CB), lambda j, fb: (0, j)),
        ],
        out_shape=[
            jax.ShapeDtypeStruct((16, C), jnp.float32),
            jax.ShapeDtypeStruct((16, C), jnp.int32),
        ],
        scratch_shapes=[pltpu.VMEM((L, CB), jnp.float32)],
        interpret=interpret,
    )(*[jnp.asarray(m) for m in mats], qt, kt)
    return w16, d16


def _sc_agg(vt, wt, dt, top_k):
    C, L = vt.shape
    info = plsc.get_sparse_core_info()
    nw = info.num_cores * info.num_subcores
    cols_per = C // nw
    mesh = plsc.VectorSubcoreMesh(core_axis_name="c", subcore_axis_name="s")

    NBUF = 2

    @functools.partial(
        pl.kernel,
        out_type=jax.ShapeDtypeStruct((C, L), jnp.float32),
        mesh=mesh,
        scratch_types=[
            pltpu.VMEM((2 * L,), jnp.float32),
            pltpu.VMEM((2 * L,), jnp.float32),
            pltpu.VMEM((L,), jnp.float32),
            pltpu.VMEM((L,), jnp.float32),
            pltpu.VMEM((cols_per, 16), jnp.float32),
            pltpu.VMEM((cols_per, 16), jnp.int32),
            pltpu.SemaphoreType.DMA,
            pltpu.SemaphoreType.DMA,
            pltpu.SemaphoreType.DMA,
            pltpu.SemaphoreType.DMA,
        ],
    )
    def body(vt_hbm, wt_hbm, dt_hbm, out_hbm, vb0, vb1, ob0, ob1,
             wall, dall, si0, si1, so0, so1):
        vbufs = [vb0, vb1]
        obufs = [ob0, ob1]
        sin = [si0, si1]
        sout = [so0, so1]
        wid = lax.axis_index("s") * info.num_cores + lax.axis_index("c")
        base_col = wid * cols_per
        pltpu.sync_copy(wt_hbm.at[pl.ds(base_col, cols_per)], wall)
        pltpu.sync_copy(dt_hbm.at[pl.ds(base_col, cols_per)], dall)

        def in_copies(c0, b):
            return (
                pltpu.make_async_copy(
                    vt_hbm.at[c0], vbufs[b].at[pl.ds(0, L)], sin[b]),
                pltpu.make_async_copy(
                    vt_hbm.at[c0], vbufs[b].at[pl.ds(L, L)], sin[b]),
            )

        for b in range(NBUF):
            for cp in in_copies(base_col + b, b):
                cp.start()

        def outer(g, carry):
            for b in range(NBUF):
                j = g * NBUF + b
                c0 = base_col + j
                for cp in in_copies(c0, b):
                    cp.wait()

                @pl.when(g > 0)
                def _():
                    pltpu.make_async_copy(
                        obufs[b], out_hbm.at[c0 - NBUF], sout[b]).wait()

                wv = wall[j]
                dv = dall[j]
                wss = [wv[i] for i in range(top_k)]
                dss = [dv[i] for i in range(top_k)]
                vb = vbufs[b]
                ob = obufs[b]

                def vec_body(v, carry2):
                    base = v * 16
                    acc = wss[0] * vb[pl.ds(base + dss[0], 16)]
                    for i in range(1, top_k):
                        acc = acc + wss[i] * vb[pl.ds(base + dss[i], 16)]
                    ob[pl.ds(base, 16)] = acc
                    return carry2

                lax.fori_loop(0, L // 16, vec_body, 0, unroll=2)
                pltpu.async_copy(ob, out_hbm.at[c0], sout[b])

                @pl.when(j + NBUF < cols_per)
                def _():
                    for cp in in_copies(c0 + NBUF, b):
                        cp.start()

            return carry

        lax.fori_loop(0, cols_per // NBUF, outer, 0)
        for b in range(NBUF):
            pltpu.make_async_copy(
                obufs[b], out_hbm.at[base_col + cols_per - NBUF + b],
                sout[b]).wait()

    return body(vt, wt, dt)


def kernel(Q, K, V):
    B, H, L, D = Q.shape
    C = B * H * D
    top_k = int(_FACTOR * math.log(L))

    qt = jnp.transpose(Q, (2, 0, 1, 3)).reshape(L, C)
    kt = jnp.transpose(K, (2, 0, 1, 3)).reshape(L, C)
    w16, d16 = _corr_topk(qt, kt, top_k)

    vt = jnp.transpose(V, (0, 1, 3, 2)).reshape(C, L)
    wt = jnp.transpose(w16, (1, 0))
    dt = jnp.transpose(d16, (1, 0))
    out_t = _sc_agg(vt, wt, dt, top_k)

    return jnp.transpose(out_t.reshape(B, H, D, L), (0, 1, 3, 2))


# speedup vs baseline: 7.3520x; 1.0322x over previous
import functools
import math

import numpy as np
import jax
import jax.numpy as jnp
from jax import lax
from jax.experimental import pallas as pl
from jax.experimental.pallas import tpu as pltpu
from jax.experimental.pallas import tpu_sc as plsc

_FACTOR = 1


@functools.lru_cache(maxsize=None)
def _dft_mats(L: int, FP: int):
    F = L // 2 + 1
    f = np.arange(FP, dtype=np.int64)[:, None]
    t = np.arange(L, dtype=np.int64)[None, :]
    ang = 2.0 * np.pi * ((f * t) % L).astype(np.float64) / L
    valid = (f < F).astype(np.float64)
    cr = (np.cos(ang) * valid).astype(np.float32)
    ci = (-np.sin(ang) * valid).astype(np.float32)
    alpha = np.where((f == 0) | (f == L // 2), 1.0, 2.0) * valid
    dr = ((np.cos(ang) * alpha / L).T).astype(np.float32)
    di = ((-np.sin(ang) * alpha / L).T).astype(np.float32)
    return cr, ci, dr, di


def _split_bf16(x):
    hi = x.astype(jnp.bfloat16)
    lo = (x - hi.astype(jnp.float32)).astype(jnp.bfloat16)
    return hi, lo


def _mm3(ah, al, bh, bl):
    f32 = jnp.float32
    return (jnp.dot(ah, bh, preferred_element_type=f32)
            + jnp.dot(ah, bl, preferred_element_type=f32)
            + jnp.dot(al, bh, preferred_element_type=f32))


def _corr_topk_body(top_k, L, CB, qth, kth,
                    mh0, mh1, mh2, mh3, mh4, mh5, mh6, mh7,
                    w_out, d_out,
                    s0, s1, s2, s3, s4, s5, s6, s7, sem):
    f32 = jnp.float32
    hbm_mats = [mh0, mh1, mh2, mh3, mh4, mh5, mh6, mh7]
    scr_mats = [s0, s1, s2, s3, s4, s5, s6, s7]

    @pl.when(pl.program_id(0) == 0)
    def _():
        cps = [pltpu.make_async_copy(src, dst, sem)
               for src, dst in zip(hbm_mats, scr_mats)]
        for cp in cps:
            cp.start()
        for cp in cps:
            cp.wait()

    crh, crl, cih, cil, drh, drl, dih, dil = scr_mats
    qh, ql = _split_bf16(qth[...])
    kh, kl = _split_bf16(kth[...])
    qr = _mm3(crh[...], crl[...], qh, ql)
    qi = _mm3(cih[...], cil[...], qh, ql)
    kr = _mm3(crh[...], crl[...], kh, kl)
    ki = _mm3(cih[...], cil[...], kh, kl)
    rr = qr * kr + qi * ki
    ri = qi * kr - qr * ki
    rrh, rrl = _split_bf16(rr)
    rih, ril = _split_bf16(ri)
    c = (_mm3(drh[...], drl[...], rrh, rrl)
         + _mm3(dih[...], dil[...], rih, ril))

    iot = lax.broadcasted_iota(jnp.int32, (L, CB), 0)
    ws, ds = [], []
    for _i in range(top_k):
        mx = jnp.max(c, axis=0, keepdims=True)
        eq = c >= mx
        idx = jnp.min(jnp.where(eq, iot, L), axis=0, keepdims=True)
        ws.append(mx)
        ds.append(idx)
        c = jnp.where(iot == idx, f32(-3.0e38), c)
    w = jnp.concatenate(ws, axis=0)
    d = jnp.concatenate(ds, axis=0)
    m = jnp.max(w, axis=0, keepdims=True)
    e = jnp.exp(w - m)
    w = e / jnp.sum(e, axis=0, keepdims=True)
    pad = 16 - top_k
    w_out[...] = jnp.concatenate([w, jnp.zeros((pad, CB), f32)], axis=0)
    d_out[...] = jnp.concatenate(
        [d, jnp.zeros((pad, CB), jnp.int32)], axis=0)


@functools.lru_cache(maxsize=None)
def _dft_mats_split(L: int, FP: int):
    import ml_dtypes
    out = []
    for m in _dft_mats(L, FP):
        hi = m.astype(ml_dtypes.bfloat16)
        lo = (m - hi.astype(np.float32)).astype(ml_dtypes.bfloat16)
        out.append(hi)
        out.append(lo)
    return tuple(out)


def _corr_topk(qth, kth, top_k, CB=128, FP=1152, interpret=False):
    L, C = qth.shape
    mats = _dft_mats_split(L, FP)
    body = functools.partial(_corr_topk_body, top_k, L, CB)
    grid = (C // CB,)
    bf16 = jnp.bfloat16
    w16, d16 = pl.pallas_call(
        body,
        grid=grid,
        in_specs=[pl.BlockSpec((L, CB), lambda j: (0, j))] * 2
        + [pl.BlockSpec(memory_space=pltpu.MemorySpace.HBM)] * 8,
        out_specs=[
            pl.BlockSpec((16, CB), lambda j: (0, j)),
            pl.BlockSpec((16, CB), lambda j: (0, j)),
        ],
        out_shape=[
            jax.ShapeDtypeStruct((16, C), jnp.float32),
            jax.ShapeDtypeStruct((16, C), jnp.int32),
        ],
        scratch_shapes=[pltpu.VMEM((FP, L), bf16)] * 4
        + [pltpu.VMEM((L, FP), bf16)] * 4
        + [pltpu.SemaphoreType.DMA],
        compiler_params=pltpu.CompilerParams(
            vmem_limit_bytes=63 * 1024 * 1024),
        interpret=interpret,
    )(qth, kth, *[jnp.asarray(m) for m in mats])
    return w16, d16


def _sc_agg(vt, wt, dt, top_k):
    C, L = vt.shape
    info = plsc.get_sparse_core_info()
    nw = info.num_cores * info.num_subcores
    cols_per = C // nw
    mesh = plsc.VectorSubcoreMesh(core_axis_name="c", subcore_axis_name="s")

    NBUF = 2

    @functools.partial(
        pl.kernel,
        out_type=jax.ShapeDtypeStruct((C, L), jnp.float32),
        mesh=mesh,
        scratch_types=[
            pltpu.VMEM((2 * L,), jnp.float32),
            pltpu.VMEM((2 * L,), jnp.float32),
            pltpu.VMEM((L,), jnp.float32),
            pltpu.VMEM((L,), jnp.float32),
            pltpu.VMEM((cols_per, 16), jnp.float32),
            pltpu.VMEM((cols_per, 16), jnp.int32),
            pltpu.SemaphoreType.DMA,
            pltpu.SemaphoreType.DMA,
            pltpu.SemaphoreType.DMA,
            pltpu.SemaphoreType.DMA,
        ],
    )
    def body(vt_hbm, wt_hbm, dt_hbm, out_hbm, vb0, vb1, ob0, ob1,
             wall, dall, si0, si1, so0, so1):
        vbufs = [vb0, vb1]
        obufs = [ob0, ob1]
        sin = [si0, si1]
        sout = [so0, so1]
        wid = lax.axis_index("s") * info.num_cores + lax.axis_index("c")
        base_col = wid * cols_per
        pltpu.sync_copy(wt_hbm.at[pl.ds(base_col, cols_per)], wall)
        pltpu.sync_copy(dt_hbm.at[pl.ds(base_col, cols_per)], dall)

        def in_copies(c0, b):
            return (
                pltpu.make_async_copy(
                    vt_hbm.at[c0], vbufs[b].at[pl.ds(0, L)], sin[b]),
                pltpu.make_async_copy(
                    vt_hbm.at[c0], vbufs[b].at[pl.ds(L, L)], sin[b]),
            )

        for b in range(NBUF):
            for cp in in_copies(base_col + b, b):
                cp.start()

        def outer(g, carry):
            for b in range(NBUF):
                j = g * NBUF + b
                c0 = base_col + j
                for cp in in_copies(c0, b):
                    cp.wait()

                @pl.when(g > 0)
                def _():
                    pltpu.make_async_copy(
                        obufs[b], out_hbm.at[c0 - NBUF], sout[b]).wait()

                wv = wall[j]
                dv = dall[j]
                wss = [wv[i] for i in range(top_k)]
                dss = [dv[i] for i in range(top_k)]
                vb = vbufs[b]
                ob = obufs[b]

                def vec_body(v, carry2):
                    base = v * 16
                    acc = wss[0] * vb[pl.ds(base + dss[0], 16)]
                    for i in range(1, top_k):
                        acc = acc + wss[i] * vb[pl.ds(base + dss[i], 16)]
                    ob[pl.ds(base, 16)] = acc
                    return carry2

                lax.fori_loop(0, L // 16, vec_body, 0, unroll=2)
                pltpu.async_copy(ob, out_hbm.at[c0], sout[b])

                @pl.when(j + NBUF < cols_per)
                def _():
                    for cp in in_copies(c0 + NBUF, b):
                        cp.start()

            return carry

        lax.fori_loop(0, cols_per // NBUF, outer, 0)
        for b in range(NBUF):
            pltpu.make_async_copy(
                obufs[b], out_hbm.at[base_col + cols_per - NBUF + b],
                sout[b]).wait()

    return body(vt, wt, dt)


def kernel(Q, K, V):
    B, H, L, D = Q.shape
    C = B * H * D
    top_k = int(_FACTOR * math.log(L))

    qt = jnp.transpose(Q, (2, 0, 1, 3)).reshape(L, C)
    kt = jnp.transpose(K, (2, 0, 1, 3)).reshape(L, C)
    w16, d16 = _corr_topk(qt, kt, top_k)

    vt = jnp.transpose(V, (0, 1, 3, 2)).reshape(C, L)
    wt = jnp.transpose(w16, (1, 0))
    dt = jnp.transpose(d16, (1, 0))
    out_t = _sc_agg(vt, wt, dt, top_k)

    return jnp.transpose(out_t.reshape(B, H, D, L), (0, 1, 3, 2))


# speedup vs baseline: 10.8440x; 1.4750x over previous
import functools
import math

import numpy as np
import jax
import jax.numpy as jnp
from jax import lax
from jax.experimental import pallas as pl
from jax.experimental.pallas import tpu as pltpu
from jax.experimental.pallas import tpu_sc as plsc

_FACTOR = 1


@functools.lru_cache(maxsize=None)
def _dft_mats(L: int, FP: int):
    F = L // 2 + 1
    f = np.arange(FP, dtype=np.int64)[:, None]
    t = np.arange(L, dtype=np.int64)[None, :]
    ang = 2.0 * np.pi * ((f * t) % L).astype(np.float64) / L
    valid = (f < F).astype(np.float64)
    cr = (np.cos(ang) * valid).astype(np.float32)
    ci = (-np.sin(ang) * valid).astype(np.float32)
    alpha = np.where((f == 0) | (f == L // 2), 1.0, 2.0) * valid
    dr = ((np.cos(ang) * alpha / L).T).astype(np.float32)
    di = ((-np.sin(ang) * alpha / L).T).astype(np.float32)
    return cr, ci, dr, di


def _split_bf16(x):
    hi = x.astype(jnp.bfloat16)
    lo = (x - hi.astype(jnp.float32)).astype(jnp.bfloat16)
    return hi, lo


def _mm3(ah, al, bh, bl):
    f32 = jnp.float32
    return (jnp.dot(ah, bh, preferred_element_type=f32)
            + jnp.dot(ah, bl, preferred_element_type=f32)
            + jnp.dot(al, bh, preferred_element_type=f32))


def _mm3t(ah, al, bh, bl):
    f32 = jnp.float32
    dn = (((0,), (0,)), ((), ()))
    return (lax.dot_general(ah, bh, dn, preferred_element_type=f32)
            + lax.dot_general(ah, bl, dn, preferred_element_type=f32)
            + lax.dot_general(al, bh, dn, preferred_element_type=f32))


def _corr_topk_body(top_k, L, FP, CB, qth, kth,
                    mh0, mh1, mh2, mh3,
                    w_out, d_out,
                    s0, s1, s2, s3, sem):
    f32 = jnp.float32
    hbm_mats = [mh0, mh1, mh2, mh3]
    scr_mats = [s0, s1, s2, s3]

    @pl.when(pl.program_id(0) == 0)
    def _():
        cps = [pltpu.make_async_copy(src, dst, sem)
               for src, dst in zip(hbm_mats, scr_mats)]
        for cp in cps:
            cp.start()
        for cp in cps:
            cp.wait()

    crh, crl, cih, cil = (s[...] for s in scr_mats)
    qh, ql = _split_bf16(qth[...])
    kh, kl = _split_bf16(kth[...])
    qr = _mm3(crh, crl, qh, ql)
    qi = _mm3(cih, cil, qh, ql)
    kr = _mm3(crh, crl, kh, kl)
    ki = _mm3(cih, cil, kh, kl)
    fidx = lax.broadcasted_iota(jnp.int32, (FP, CB), 0)
    a = jnp.where((fidx == 0) | (fidx == L // 2), 1.0, 2.0).astype(f32)
    a = jnp.where(fidx <= L // 2, a, 0.0) * f32(1.0 / L)
    rr = (qr * kr + qi * ki) * a
    ri = (qi * kr - qr * ki) * a
    rrh, rrl = _split_bf16(rr)
    rih, ril = _split_bf16(ri)
    c = _mm3t(crh, crl, rrh, rrl) + _mm3t(cih, cil, rih, ril)

    iot = lax.broadcasted_iota(jnp.int32, (L, CB), 0)
    ws, ds = [], []
    for _i in range(top_k):
        mx = jnp.max(c, axis=0, keepdims=True)
        eq = c >= mx
        idx = jnp.min(jnp.where(eq, iot, L), axis=0, keepdims=True)
        ws.append(mx)
        ds.append(idx)
        c = jnp.where(iot == idx, f32(-3.0e38), c)
    w = jnp.concatenate(ws, axis=0)
    d = jnp.concatenate(ds, axis=0)
    m = jnp.max(w, axis=0, keepdims=True)
    e = jnp.exp(w - m)
    w = e / jnp.sum(e, axis=0, keepdims=True)
    pad = 16 - top_k
    w_out[...] = jnp.concatenate([w, jnp.zeros((pad, CB), f32)], axis=0)
    d_out[...] = jnp.concatenate(
        [d, jnp.zeros((pad, CB), jnp.int32)], axis=0)


@functools.lru_cache(maxsize=None)
def _dft_mats_split(L: int, FP: int):
    import ml_dtypes
    out = []
    for m in _dft_mats(L, FP)[:2]:
        hi = m.astype(ml_dtypes.bfloat16)
        lo = (m - hi.astype(np.float32)).astype(ml_dtypes.bfloat16)
        out.append(hi)
        out.append(lo)
    return tuple(out)


def _corr_topk(qth, kth, top_k, CB=256, FP=1152, interpret=False):
    L, C = qth.shape
    mats = _dft_mats_split(L, FP)
    body = functools.partial(_corr_topk_body, top_k, L, FP, CB)
    grid = (C // CB,)
    bf16 = jnp.bfloat16
    w16, d16 = pl.pallas_call(
        body,
        grid=grid,
        in_specs=[pl.BlockSpec((L, CB), lambda j: (0, j))] * 2
        + [pl.BlockSpec(memory_space=pltpu.MemorySpace.HBM)] * 4,
        out_specs=[
            pl.BlockSpec((16, CB), lambda j: (0, j)),
            pl.BlockSpec((16, CB), lambda j: (0, j)),
        ],
        out_shape=[
            jax.ShapeDtypeStruct((16, C), jnp.float32),
            jax.ShapeDtypeStruct((16, C), jnp.int32),
        ],
        scratch_shapes=[pltpu.VMEM((FP, L), bf16)] * 4
        + [pltpu.SemaphoreType.DMA],
        compiler_params=pltpu.CompilerParams(
            vmem_limit_bytes=63 * 1024 * 1024),
        interpret=interpret,
    )(qth, kth, *[jnp.asarray(m) for m in mats])
    return w16, d16


def _sc_agg(vt, wt, dt, top_k):
    C, L = vt.shape
    info = plsc.get_sparse_core_info()
    nw = info.num_cores * info.num_subcores
    cols_per = C // nw
    mesh = plsc.VectorSubcoreMesh(core_axis_name="c", subcore_axis_name="s")

    NBUF = 2

    @functools.partial(
        pl.kernel,
        out_type=jax.ShapeDtypeStruct((C, L), jnp.float32),
        mesh=mesh,
        scratch_types=[
            pltpu.VMEM((2 * L,), jnp.float32),
            pltpu.VMEM((2 * L,), jnp.float32),
            pltpu.VMEM((L,), jnp.float32),
            pltpu.VMEM((L,), jnp.float32),
            pltpu.VMEM((cols_per, 16), jnp.float32),
            pltpu.VMEM((cols_per, 16), jnp.int32),
            pltpu.SemaphoreType.DMA,
            pltpu.SemaphoreType.DMA,
            pltpu.SemaphoreType.DMA,
            pltpu.SemaphoreType.DMA,
        ],
    )
    def body(vt_hbm, wt_hbm, dt_hbm, out_hbm, vb0, vb1, ob0, ob1,
             wall, dall, si0, si1, so0, so1):
        vbufs = [vb0, vb1]
        obufs = [ob0, ob1]
        sin = [si0, si1]
        sout = [so0, so1]
        wid = lax.axis_index("s") * info.num_cores + lax.axis_index("c")
        base_col = wid * cols_per
        pltpu.sync_copy(wt_hbm.at[pl.ds(base_col, cols_per)], wall)
        pltpu.sync_copy(dt_hbm.at[pl.ds(base_col, cols_per)], dall)

        def in_copies(c0, b):
            return (
                pltpu.make_async_copy(
                    vt_hbm.at[c0], vbufs[b].at[pl.ds(0, L)], sin[b]),
                pltpu.make_async_copy(
                    vt_hbm.at[c0], vbufs[b].at[pl.ds(L, L)], sin[b]),
            )

        for b in range(NBUF):
            for cp in in_copies(base_col + b, b):
                cp.start()

        def outer(g, carry):
            for b in range(NBUF):
                j = g * NBUF + b
                c0 = base_col + j
                for cp in in_copies(c0, b):
                    cp.wait()

                @pl.when(g > 0)
                def _():
                    pltpu.make_async_copy(
                        obufs[b], out_hbm.at[c0 - NBUF], sout[b]).wait()

                wv = wall[j]
                dv = dall[j]
                wss = [wv[i] for i in range(top_k)]
                dss = [dv[i] for i in range(top_k)]
                vb = vbufs[b]
                ob = obufs[b]

                def vec_body(v, carry2):
                    base = v * 16
                    acc = wss[0] * vb[pl.ds(base + dss[0], 16)]
                    for i in range(1, top_k):
                        acc = acc + wss[i] * vb[pl.ds(base + dss[i], 16)]
                    ob[pl.ds(base, 16)] = acc
                    return carry2

                lax.fori_loop(0, L // 16, vec_body, 0, unroll=2)
                pltpu.async_copy(ob, out_hbm.at[c0], sout[b])

                @pl.when(j + NBUF < cols_per)
                def _():
                    for cp in in_copies(c0 + NBUF, b):
                        cp.start()

            return carry

        lax.fori_loop(0, cols_per // NBUF, outer, 0)
        for b in range(NBUF):
            pltpu.make_async_copy(
                obufs[b], out_hbm.at[base_col + cols_per - NBUF + b],
                sout[b]).wait()

    return body(vt, wt, dt)


def kernel(Q, K, V):
    B, H, L, D = Q.shape
    C = B * H * D
    top_k = int(_FACTOR * math.log(L))

    qt = jnp.transpose(Q, (2, 0, 1, 3)).reshape(L, C)
    kt = jnp.transpose(K, (2, 0, 1, 3)).reshape(L, C)
    w16, d16 = _corr_topk(qt, kt, top_k)

    vt = jnp.transpose(V, (0, 1, 3, 2)).reshape(C, L)
    wt = jnp.transpose(w16, (1, 0))
    dt = jnp.transpose(d16, (1, 0))
    out_t = _sc_agg(vt, wt, dt, top_k)

    return jnp.transpose(out_t.reshape(B, H, D, L), (0, 1, 3, 2))
